# baseline (speedup 1.0000x reference)
import functools
import math

import jax
import jax.numpy as jnp
from jax import lax
from jax.experimental import pallas as pl
from jax.experimental.pallas import tpu as pltpu
from jax.experimental.pallas import tpu_sc as plsc

D_MODEL = 64
SCALE = math.sqrt(D_MODEL)
LANES = 16

_info = plsc.get_sparse_core_info()
NUM_CORES = _info.num_cores
NUM_SUBCORES = _info.num_subcores
NUM_WORKERS = NUM_CORES * NUM_SUBCORES

CHUNK = 128


def _make_lookup(batch, d_model):
    assert batch % (NUM_WORKERS * CHUNK) == 0
    b_per_w = batch // NUM_WORKERS
    n_chunks = b_per_w // CHUNK
    mesh = plsc.VectorSubcoreMesh(core_axis_name="c", subcore_axis_name="s")

    @functools.partial(
        pl.kernel,
        mesh=mesh,
        compiler_params=pltpu.CompilerParams(use_tc_tiling_on_sc=False),
        out_type=jax.ShapeDtypeStruct((batch, d_model), jnp.float32),
        scratch_types=[
            pltpu.VMEM((CHUNK,), jnp.int32),
            pltpu.VMEM((CHUNK, d_model), jnp.float32),
            pltpu.SemaphoreType.DMA,
        ],
    )
    def lookup(x_hbm, table_hbm, out_hbm, idx_v, rows_v, sem):
        wid = lax.axis_index("s") * NUM_CORES + lax.axis_index("c")
        w_base = wid * b_per_w

        def chunk_body(g, carry):
            base = w_base + g * CHUNK
            pltpu.sync_copy(x_hbm.at[pl.ds(base, CHUNK)], idx_v)
            pltpu.async_copy(table_hbm.at[idx_v], rows_v, sem).wait()

            def row_body(r, c):
                for j in range(d_model // LANES):
                    sl = pl.ds(j * LANES, LANES)
                    rows_v[r, sl] = rows_v[r, sl] * SCALE
                return c

            lax.fori_loop(0, CHUNK, row_body, 0)
            pltpu.sync_copy(rows_v, out_hbm.at[pl.ds(base, CHUNK)])
            return carry

        lax.fori_loop(0, n_chunks, chunk_body, 0)

    return lookup


def kernel(x, table):
    batch = x.shape[0] * x.shape[1]
    flat_idx = x.reshape(batch).astype(jnp.int32)
    out = _make_lookup(batch, table.shape[1])(flat_idx, table)
    return out.reshape(x.shape[0], x.shape[1], table.shape[1])


# speedup vs baseline: 1.2794x; 1.2794x over previous
import functools
import math

import jax
import jax.numpy as jnp
from jax import lax
from jax.experimental import pallas as pl
from jax.experimental.pallas import tpu as pltpu
from jax.experimental.pallas import tpu_sc as plsc

D_MODEL = 64
SCALE = math.sqrt(D_MODEL)
LANES = 16

_info = plsc.get_sparse_core_info()
NUM_CORES = _info.num_cores
NUM_SUBCORES = _info.num_subcores
NUM_WORKERS = NUM_CORES * NUM_SUBCORES

CHUNK = 128
NBUF = 4


def _make_lookup(batch, d_model):
    assert batch % (NUM_WORKERS * CHUNK) == 0
    b_per_w = batch // NUM_WORKERS
    n_chunks = b_per_w // CHUNK
    n_groups = n_chunks // NBUF
    assert n_chunks % NBUF == 0
    assert n_groups >= 4 and n_groups % 2 == 0
    n_pairs = (n_groups - 2) // 2

    mesh = plsc.VectorSubcoreMesh(core_axis_name="c", subcore_axis_name="s")

    @functools.partial(
        pl.kernel,
        mesh=mesh,
        compiler_params=pltpu.CompilerParams(use_tc_tiling_on_sc=False),
        out_type=jax.ShapeDtypeStruct((batch, d_model), jnp.float32),
        scratch_types=[
            pltpu.VMEM((n_chunks, CHUNK), jnp.int32),
            pltpu.VMEM((2 * NBUF, CHUNK, d_model), jnp.float32),
            pltpu.SemaphoreType.DMA((2 * NBUF,)),
            pltpu.SemaphoreType.DMA((2 * NBUF,)),
        ],
    )
    def lookup(x_hbm, table_hbm, out_hbm, idx_v, rows_v, gsem, ssem):
        wid = lax.axis_index("s") * NUM_CORES + lax.axis_index("c")
        w_base = wid * b_per_w

        pltpu.sync_copy(x_hbm.at[wid], idx_v)

        def gather_start(g, s):
            pltpu.async_copy(table_hbm.at[idx_v.at[g]], rows_v.at[s],
                             gsem.at[s])

        def gather_wait(g, s):
            pltpu.make_async_copy(table_hbm.at[idx_v.at[g]], rows_v.at[s],
                                  gsem.at[s]).wait()

        def scatter_start(g, s):
            pltpu.async_copy(rows_v.at[s],
                             out_hbm.at[pl.ds(w_base + g * CHUNK, CHUNK)],
                             ssem.at[s])

        def scatter_wait(g, s):
            pltpu.make_async_copy(rows_v.at[s],
                                  out_hbm.at[pl.ds(w_base + g * CHUNK, CHUNK)],
                                  ssem.at[s]).wait()

        def scale(s):
            def row_body(r, c):
                for j in range(d_model // LANES):
                    sl = pl.ds(j * LANES, LANES)
                    rows_v[s, r, sl] = rows_v[s, r, sl] * SCALE
                return c
            lax.fori_loop(0, CHUNK, row_body, 0, unroll=2)

        for b in range(NBUF):
            gather_start(b, b)

        for b in range(NBUF):
            gather_wait(b, b)
            scale(b)
            scatter_start(b, b)
            gather_start(NBUF + b, NBUF + b)

        def pair_body(i, carry):
            grp0 = 1 + 2 * i
            for h in range(2):
                grp = grp0 + h
                p = (1 + h) % 2
                q = 1 - p
                for b in range(NBUF):
                    s = p * NBUF + b
                    sq = q * NBUF + b
                    g = grp * NBUF + b
                    gather_wait(g, s)
                    scale(s)
                    scatter_start(g, s)
                    scatter_wait((grp - 1) * NBUF + b, sq)
                    gather_start((grp + 1) * NBUF + b, sq)
            return carry

        lax.fori_loop(0, n_pairs, pair_body, 0)

        grp = n_groups - 1
        for b in range(NBUF):
            s = NBUF + b
            g = grp * NBUF + b
            gather_wait(g, s)
            scale(s)
            scatter_start(g, s)

        for b in range(NBUF):
            scatter_wait((n_groups - 2) * NBUF + b, b)
            scatter_wait((n_groups - 1) * NBUF + b, NBUF + b)

    return lookup


def kernel(x, table):
    batch = x.shape[0] * x.shape[1]
    b_per_w = batch // NUM_WORKERS
    n_chunks = b_per_w // CHUNK
    flat_idx = x.reshape(NUM_WORKERS, n_chunks, CHUNK).astype(jnp.int32)
    out = _make_lookup(batch, table.shape[1])(flat_idx, table)
    return out.reshape(x.shape[0], x.shape[1], table.shape[1])
